# baseline (speedup 1.0000x reference)
import functools

import jax
import jax.numpy as jnp
from jax import lax
from jax.experimental import pallas as pl
from jax.experimental.pallas import tpu as pltpu
from jax.experimental.pallas import tpu_sc as plsc

NUM_E = 1_000_000
D = 64
NC = 2
NS = 16
L = 16
NW = NC * NS
RPW = NUM_E // NW
CHUNK = 625
NCHUNK = RPW // CHUNK

def _scan_body(head_hbm, q_hbm, ent_hbm, vals_out, idx_out,
               buf0, buf1, hbuf, qbuf, cv, ci, sem0, sem1):
    wid = lax.axis_index("s") * NC + lax.axis_index("c")
    base = wid * RPW

    pltpu.sync_copy(head_hbm, hbuf)
    pltpu.sync_copy(q_hbm, qbuf)

    h0 = hbuf[pl.ds(0, L)]
    h1 = hbuf[pl.ds(L, L)]
    h2 = hbuf[pl.ds(2 * L, L)]
    h3 = hbuf[pl.ds(3 * L, L)]
    q0 = qbuf[pl.ds(0, L)]
    q1 = qbuf[pl.ds(L, L)]
    q2 = qbuf[pl.ds(2 * L, L)]
    q3 = qbuf[pl.ds(3 * L, L)]
    wq0 = h0 * q0 - h2 * q2
    wq1 = h1 * q1 - h3 * q3
    wq2 = h2 * q0 + h0 * q2
    wq3 = h3 * q1 + h1 * q3

    def scan_chunk(buf, r0, bv, bi):
        def row_body(r, c):
            bv, bi = c
            t = (buf[r, pl.ds(0, L)] * wq0
                 + buf[r, pl.ds(L, L)] * wq1
                 + buf[r, pl.ds(2 * L, L)] * wq2
                 + buf[r, pl.ds(3 * L, L)] * wq3)
            s = jnp.sum(t)
            better = s > bv
            bv = jnp.where(better, s, bv)
            bi = jnp.where(better, r0 + r, bi)
            return bv, bi
        return lax.fori_loop(0, CHUNK, row_body, (bv, bi))

    pltpu.async_copy(ent_hbm.at[pl.ds(base, CHUNK)], buf0, sem0)

    def outer(i, carry):
        bv, bi = carry
        r0 = base + (2 * i) * CHUNK
        r1 = r0 + CHUNK
        pltpu.async_copy(ent_hbm.at[pl.ds(r1, CHUNK)], buf1, sem1)
        pltpu.make_async_copy(ent_hbm.at[pl.ds(r0, CHUNK)], buf0, sem0).wait()
        bv, bi = scan_chunk(buf0, r0, bv, bi)

        @pl.when(i + 1 < NCHUNK // 2)
        def _():
            pltpu.async_copy(ent_hbm.at[pl.ds(r1 + CHUNK, CHUNK)], buf0, sem0)

        pltpu.make_async_copy(ent_hbm.at[pl.ds(r1, CHUNK)], buf1, sem1).wait()
        bv, bi = scan_chunk(buf1, r1, bv, bi)
        return bv, bi

    bv, bi = lax.fori_loop(0, NCHUNK // 2, outer,
                           (jnp.float32(-jnp.inf), jnp.int32(0)))

    cv[...] = jnp.full((L,), bv, jnp.float32)
    ci[...] = jnp.full((L,), bi, jnp.int32)
    pltpu.sync_copy(cv, vals_out.at[wid])
    pltpu.sync_copy(ci, idx_out.at[wid])


@functools.lru_cache(maxsize=None)
def _build_scan():
    mesh = plsc.VectorSubcoreMesh(core_axis_name="c", subcore_axis_name="s",
                                  num_cores=NC, num_subcores=NS)
    return pl.kernel(
        _scan_body,
        out_type=(
            jax.ShapeDtypeStruct((NW, L), jnp.float32),
            jax.ShapeDtypeStruct((NW, L), jnp.int32),
        ),
        mesh=mesh,
        scratch_types=[
            pltpu.VMEM((CHUNK, D), jnp.float32),
            pltpu.VMEM((CHUNK, D), jnp.float32),
            pltpu.VMEM((D,), jnp.float32),
            pltpu.VMEM((D,), jnp.float32),
            pltpu.VMEM((L,), jnp.float32),
            pltpu.VMEM((L,), jnp.int32),
            pltpu.SemaphoreType.DMA,
            pltpu.SemaphoreType.DMA,
        ],
        compiler_params=pltpu.CompilerParams(use_tc_tiling_on_sc=False,
                                             needs_layout_passes=False),
    )


def _merge_body(vals_ref, idx_ref, ent_ref, out_ref, row_buf, sem):
    vals = vals_ref[...]
    idx = idx_ref[...]
    m = jnp.max(vals)
    big = jnp.int32(jnp.iinfo(jnp.int32).max)
    best = jnp.min(jnp.where(vals >= m, idx, big))
    base8 = pl.multiple_of((best // 8) * 8, 8)
    copy = pltpu.make_async_copy(ent_ref.at[pl.ds(base8, 8)], row_buf, sem)
    copy.start()
    copy.wait()
    rows = row_buf[...]
    rsel = jax.lax.broadcasted_iota(jnp.int32, (8, D), 0) == (best % 8)
    out_ref[...] = jnp.sum(jnp.where(rsel, rows, 0.0), axis=0)


_merge = pl.pallas_call(
    _merge_body,
    out_shape=jax.ShapeDtypeStruct((D,), jnp.float32),
    in_specs=[
        pl.BlockSpec(memory_space=pltpu.VMEM),
        pl.BlockSpec(memory_space=pltpu.VMEM),
        pl.BlockSpec(memory_space=pltpu.MemorySpace.HBM),
    ],
    scratch_shapes=[
        pltpu.VMEM((8, D), jnp.float32),
        pltpu.SemaphoreType.DMA,
    ],
)


def kernel(head_entity, question_embedding, entity_embeddings):
    vals, idx = _build_scan()(head_entity, question_embedding,
                              entity_embeddings)
    return _merge(vals, idx, entity_embeddings)


# speedup vs baseline: 1.0169x; 1.0169x over previous
import functools

import jax
import jax.numpy as jnp
from jax import lax
from jax.experimental import pallas as pl
from jax.experimental.pallas import tpu as pltpu
from jax.experimental.pallas import tpu_sc as plsc

NUM_E = 1_000_000
D = 64
NC = 2
NS = 16
L = 16
NW = NC * NS
RPW = NUM_E // NW
CHUNK = 625
NCHUNK = RPW // CHUNK

def _scan_body(head_hbm, q_hbm, ent_hbm, vals_out, idx_out,
               buf0, buf1, hbuf, qbuf, cv, ci, sem0, sem1):
    wid = lax.axis_index("s") * NC + lax.axis_index("c")
    base = wid * RPW

    pltpu.sync_copy(head_hbm, hbuf)
    pltpu.sync_copy(q_hbm, qbuf)

    h0 = hbuf[pl.ds(0, L)]
    h1 = hbuf[pl.ds(L, L)]
    h2 = hbuf[pl.ds(2 * L, L)]
    h3 = hbuf[pl.ds(3 * L, L)]
    q0 = qbuf[pl.ds(0, L)]
    q1 = qbuf[pl.ds(L, L)]
    q2 = qbuf[pl.ds(2 * L, L)]
    q3 = qbuf[pl.ds(3 * L, L)]
    wq0 = h0 * q0 - h2 * q2
    wq1 = h1 * q1 - h3 * q3
    wq2 = h2 * q0 + h0 * q2
    wq3 = h3 * q1 + h1 * q3

    def scan_chunk(buf, r0, bv, bi):
        @plsc.parallel_loop(0, CHUNK, step=1, unroll=8, carry=(bv, bi))
        def row_body(r, c):
            bv, bi = c
            t0 = buf[r, pl.ds(0, L)] * wq0
            t1 = buf[r, pl.ds(L, L)] * wq1
            t2 = buf[r, pl.ds(2 * L, L)] * wq2
            t3 = buf[r, pl.ds(3 * L, L)] * wq3
            s = jnp.sum((t0 + t1) + (t2 + t3))
            ridx = r0 + r
            better = (s > bv) | ((s == bv) & (ridx < bi))
            bv = jnp.where(better, s, bv)
            bi = jnp.where(better, ridx, bi)
            return bv, bi
        return row_body

    pltpu.async_copy(ent_hbm.at[pl.ds(base, CHUNK)], buf0, sem0)

    def outer(i, carry):
        bv, bi = carry
        r0 = base + (2 * i) * CHUNK
        r1 = r0 + CHUNK
        pltpu.async_copy(ent_hbm.at[pl.ds(r1, CHUNK)], buf1, sem1)
        pltpu.make_async_copy(ent_hbm.at[pl.ds(r0, CHUNK)], buf0, sem0).wait()
        bv, bi = scan_chunk(buf0, r0, bv, bi)

        @pl.when(i + 1 < NCHUNK // 2)
        def _():
            pltpu.async_copy(ent_hbm.at[pl.ds(r1 + CHUNK, CHUNK)], buf0, sem0)

        pltpu.make_async_copy(ent_hbm.at[pl.ds(r1, CHUNK)], buf1, sem1).wait()
        bv, bi = scan_chunk(buf1, r1, bv, bi)
        return bv, bi

    bv, bi = lax.fori_loop(0, NCHUNK // 2, outer,
                           (jnp.float32(-jnp.inf), jnp.int32(0)))

    cv[...] = jnp.full((L,), bv, jnp.float32)
    ci[...] = jnp.full((L,), bi, jnp.int32)
    pltpu.sync_copy(cv, vals_out.at[wid])
    pltpu.sync_copy(ci, idx_out.at[wid])


@functools.lru_cache(maxsize=None)
def _build_scan():
    mesh = plsc.VectorSubcoreMesh(core_axis_name="c", subcore_axis_name="s",
                                  num_cores=NC, num_subcores=NS)
    return pl.kernel(
        _scan_body,
        out_type=(
            jax.ShapeDtypeStruct((NW, L), jnp.float32),
            jax.ShapeDtypeStruct((NW, L), jnp.int32),
        ),
        mesh=mesh,
        scratch_types=[
            pltpu.VMEM((CHUNK, D), jnp.float32),
            pltpu.VMEM((CHUNK, D), jnp.float32),
            pltpu.VMEM((D,), jnp.float32),
            pltpu.VMEM((D,), jnp.float32),
            pltpu.VMEM((L,), jnp.float32),
            pltpu.VMEM((L,), jnp.int32),
            pltpu.SemaphoreType.DMA,
            pltpu.SemaphoreType.DMA,
        ],
        compiler_params=pltpu.CompilerParams(use_tc_tiling_on_sc=False,
                                             needs_layout_passes=False),
    )


def _merge_body(vals_ref, idx_ref, ent_ref, out_ref, row_buf, sem):
    vals = vals_ref[...]
    idx = idx_ref[...]
    m = jnp.max(vals)
    big = jnp.int32(jnp.iinfo(jnp.int32).max)
    best = jnp.min(jnp.where(vals >= m, idx, big))
    base8 = pl.multiple_of((best // 8) * 8, 8)
    copy = pltpu.make_async_copy(ent_ref.at[pl.ds(base8, 8)], row_buf, sem)
    copy.start()
    copy.wait()
    rows = row_buf[...]
    rsel = jax.lax.broadcasted_iota(jnp.int32, (8, D), 0) == (best % 8)
    out_ref[...] = jnp.sum(jnp.where(rsel, rows, 0.0), axis=0)


_merge = pl.pallas_call(
    _merge_body,
    out_shape=jax.ShapeDtypeStruct((D,), jnp.float32),
    in_specs=[
        pl.BlockSpec(memory_space=pltpu.VMEM),
        pl.BlockSpec(memory_space=pltpu.VMEM),
        pl.BlockSpec(memory_space=pltpu.MemorySpace.HBM),
    ],
    scratch_shapes=[
        pltpu.VMEM((8, D), jnp.float32),
        pltpu.SemaphoreType.DMA,
    ],
)


def kernel(head_entity, question_embedding, entity_embeddings):
    vals, idx = _build_scan()(head_entity, question_embedding,
                              entity_embeddings)
    return _merge(vals, idx, entity_embeddings)


# speedup vs baseline: 1.5199x; 1.4946x over previous
import functools

import jax
import jax.numpy as jnp
from jax import lax
from jax.experimental import pallas as pl
from jax.experimental.pallas import tpu as pltpu
from jax.experimental.pallas import tpu_sc as plsc

NUM_E = 1_000_000
D = 64
NC = 2
NS = 16
L = 16
NW = NC * NS
CHUNK = 400
NCHUNK = NUM_E // CHUNK


def _scan_body(head_hbm, q_hbm, ent_hbm, vals_out, idx_out,
               buf0, buf1, hbuf, qbuf, cv, ci, sem0, sem1):
    wid = lax.axis_index("s") * NC + lax.axis_index("c")
    nk = (NCHUNK - 1 - wid) // NW + 1

    pltpu.sync_copy(head_hbm, hbuf)
    pltpu.sync_copy(q_hbm, qbuf)

    h0 = hbuf[pl.ds(0, L)]
    h1 = hbuf[pl.ds(L, L)]
    h2 = hbuf[pl.ds(2 * L, L)]
    h3 = hbuf[pl.ds(3 * L, L)]
    q0 = qbuf[pl.ds(0, L)]
    q1 = qbuf[pl.ds(L, L)]
    q2 = qbuf[pl.ds(2 * L, L)]
    q3 = qbuf[pl.ds(3 * L, L)]
    wq0 = h0 * q0 - h2 * q2
    wq1 = h1 * q1 - h3 * q3
    wq2 = h2 * q0 + h0 * q2
    wq3 = h3 * q1 + h1 * q3

    def row0_of(k):
        return (wid + k * NW) * CHUNK

    def start(buf, sem, k):
        pltpu.async_copy(ent_hbm.at[pl.ds(row0_of(k), CHUNK)], buf, sem)

    def wait(buf, sem, k):
        pltpu.make_async_copy(ent_hbm.at[pl.ds(row0_of(k), CHUNK)],
                              buf, sem).wait()

    def scan_chunk(buf, r0, carry):
        @plsc.parallel_loop(0, CHUNK, step=1, unroll=8, carry=carry)
        def row_body(r, c):
            bv, bi = c
            t0 = buf[r, pl.ds(0, L)] * wq0
            t1 = buf[r, pl.ds(L, L)] * wq1
            t2 = buf[r, pl.ds(2 * L, L)] * wq2
            t3 = buf[r, pl.ds(3 * L, L)] * wq3
            s = jnp.sum((t0 + t1) + (t2 + t3))
            ridx = r0 + r
            better = (s > bv) | ((s == bv) & (ridx < bi))
            bv = jnp.where(better, s, bv)
            bi = jnp.where(better, ridx, bi)
            return bv, bi
        return row_body

    start(buf0, sem0, 0)

    def outer(i2, carry):
        k0 = 2 * i2

        @pl.when(k0 + 1 < nk)
        def _():
            start(buf1, sem1, k0 + 1)

        wait(buf0, sem0, k0)
        carry = scan_chunk(buf0, row0_of(k0), carry)

        @pl.when(k0 + 2 < nk)
        def _():
            start(buf0, sem0, k0 + 2)

        def second(c):
            wait(buf1, sem1, k0 + 1)
            return scan_chunk(buf1, row0_of(k0 + 1), c)

        carry = lax.cond(k0 + 1 < nk, second, lambda c: c, carry)
        return carry

    n_outer = (nk + 1) // 2
    bv, bi = lax.fori_loop(0, n_outer, outer,
                           (jnp.float32(-jnp.inf), jnp.int32(0)))

    for i in range(8):
        cv[i] = jnp.full((L,), bv, jnp.float32)
        ci[i] = jnp.full((L,), bi, jnp.int32)
    pltpu.sync_copy(cv, vals_out.at[pl.ds(wid * 8, 8)])
    pltpu.sync_copy(ci, idx_out.at[pl.ds(wid * 8, 8)])


@functools.lru_cache(maxsize=None)
def _build_scan():
    mesh = plsc.VectorSubcoreMesh(core_axis_name="c", subcore_axis_name="s",
                                  num_cores=NC, num_subcores=NS)
    return pl.kernel(
        _scan_body,
        out_type=(
            jax.ShapeDtypeStruct((NW * 8, L), jnp.float32),
            jax.ShapeDtypeStruct((NW * 8, L), jnp.int32),
        ),
        mesh=mesh,
        scratch_types=[
            pltpu.VMEM((CHUNK, D), jnp.float32),
            pltpu.VMEM((CHUNK, D), jnp.float32),
            pltpu.VMEM((D,), jnp.float32),
            pltpu.VMEM((D,), jnp.float32),
            pltpu.VMEM((8, L), jnp.float32),
            pltpu.VMEM((8, L), jnp.int32),
            pltpu.SemaphoreType.DMA,
            pltpu.SemaphoreType.DMA,
        ],
        compiler_params=pltpu.CompilerParams(needs_layout_passes=False),
    )


def _merge_body(vals_ref, idx_ref, ent_ref, out_ref, row_buf, sem):
    vals = vals_ref[...]
    idx = idx_ref[...]
    m = jnp.max(vals)
    big = jnp.int32(jnp.iinfo(jnp.int32).max)
    best = jnp.min(jnp.where(vals >= m, idx, big))
    base8 = pl.multiple_of((best // 8) * 8, 8)
    copy = pltpu.make_async_copy(ent_ref.at[pl.ds(base8, 8)], row_buf, sem)
    copy.start()
    copy.wait()
    rows = row_buf[...]
    rsel = jax.lax.broadcasted_iota(jnp.int32, (8, D), 0) == (best % 8)
    out_ref[...] = jnp.sum(jnp.where(rsel, rows, 0.0), axis=0)


_merge = pl.pallas_call(
    _merge_body,
    out_shape=jax.ShapeDtypeStruct((D,), jnp.float32),
    in_specs=[
        pl.BlockSpec(memory_space=pltpu.VMEM),
        pl.BlockSpec(memory_space=pltpu.VMEM),
        pl.BlockSpec(memory_space=pltpu.MemorySpace.HBM),
    ],
    scratch_shapes=[
        pltpu.VMEM((8, D), jnp.float32),
        pltpu.SemaphoreType.DMA,
    ],
)


def kernel(head_entity, question_embedding, entity_embeddings):
    vals, idx = _build_scan()(head_entity, question_embedding,
                              entity_embeddings)
    return _merge(vals, idx, entity_embeddings)


# speedup vs baseline: 1.5941x; 1.0488x over previous
import functools

import jax
import jax.numpy as jnp
from jax import lax
from jax.experimental import pallas as pl
from jax.experimental.pallas import tpu as pltpu
from jax.experimental.pallas import tpu_sc as plsc

NUM_E = 1_000_000
D = 64
NC = 2
NS = 16
L = 16
NW = NC * NS
CHUNK = 320
NCHUNK = NUM_E // CHUNK


def _scan_body(head_hbm, q_hbm, ent_hbm, vals_out, idx_out,
               buf0, buf1, buf2, hbuf, qbuf, cv, ci, sem0, sem1, sem2):
    wid = lax.axis_index("s") * NC + lax.axis_index("c")
    nk = (NCHUNK - 1 - wid) // NW + 1

    pltpu.sync_copy(head_hbm, hbuf)
    pltpu.sync_copy(q_hbm, qbuf)

    h0 = hbuf[pl.ds(0, L)]
    h1 = hbuf[pl.ds(L, L)]
    h2 = hbuf[pl.ds(2 * L, L)]
    h3 = hbuf[pl.ds(3 * L, L)]
    q0 = qbuf[pl.ds(0, L)]
    q1 = qbuf[pl.ds(L, L)]
    q2 = qbuf[pl.ds(2 * L, L)]
    q3 = qbuf[pl.ds(3 * L, L)]
    wq0 = h0 * q0 - h2 * q2
    wq1 = h1 * q1 - h3 * q3
    wq2 = h2 * q0 + h0 * q2
    wq3 = h3 * q1 + h1 * q3

    def row0_of(k):
        return (wid + k * NW) * CHUNK

    def start(buf, sem, k):
        pltpu.async_copy(ent_hbm.at[pl.ds(row0_of(k), CHUNK)], buf, sem)

    def wait(buf, sem, k):
        pltpu.make_async_copy(ent_hbm.at[pl.ds(row0_of(k), CHUNK)],
                              buf, sem).wait()

    def scan_chunk(buf, r0, carry):
        @plsc.parallel_loop(0, CHUNK, step=1, unroll=8, carry=carry)
        def row_body(r, c):
            bv, bi = c
            t0 = buf[r, pl.ds(0, L)] * wq0
            t1 = buf[r, pl.ds(L, L)] * wq1
            t2 = buf[r, pl.ds(2 * L, L)] * wq2
            t3 = buf[r, pl.ds(3 * L, L)] * wq3
            s = jnp.sum((t0 + t1) + (t2 + t3))
            ridx = r0 + r
            better = (s > bv) | ((s == bv) & (ridx < bi))
            bv = jnp.where(better, s, bv)
            bi = jnp.where(better, ridx, bi)
            return bv, bi
        return row_body

    bufs = (buf0, buf1, buf2)
    sems = (sem0, sem1, sem2)

    for s in range(3):
        start(bufs[s], sems[s], s)

    def outer(i3, carry):
        k0 = 3 * i3
        for s in range(3):
            buf, sem = bufs[s], sems[s]

            def do_stage(c, k=k0 + s, buf=buf, sem=sem):
                wait(buf, sem, k)
                c = scan_chunk(buf, row0_of(k), c)

                @pl.when(k + 3 < nk)
                def _():
                    start(buf, sem, k + 3)

                return c

            carry = lax.cond(k0 + s < nk, do_stage, lambda c: c, carry)
        return carry

    n_outer = (nk + 2) // 3
    bv, bi = lax.fori_loop(0, n_outer, outer,
                           (jnp.float32(-jnp.inf), jnp.int32(0)))

    for i in range(8):
        cv[i] = jnp.full((L,), bv, jnp.float32)
        ci[i] = jnp.full((L,), bi, jnp.int32)
    pltpu.sync_copy(cv, vals_out.at[pl.ds(wid * 8, 8)])
    pltpu.sync_copy(ci, idx_out.at[pl.ds(wid * 8, 8)])


@functools.lru_cache(maxsize=None)
def _build_scan():
    mesh = plsc.VectorSubcoreMesh(core_axis_name="c", subcore_axis_name="s",
                                  num_cores=NC, num_subcores=NS)
    return pl.kernel(
        _scan_body,
        out_type=(
            jax.ShapeDtypeStruct((NW * 8, L), jnp.float32),
            jax.ShapeDtypeStruct((NW * 8, L), jnp.int32),
        ),
        mesh=mesh,
        scratch_types=[
            pltpu.VMEM((CHUNK, D), jnp.float32),
            pltpu.VMEM((CHUNK, D), jnp.float32),
            pltpu.VMEM((CHUNK, D), jnp.float32),
            pltpu.VMEM((D,), jnp.float32),
            pltpu.VMEM((D,), jnp.float32),
            pltpu.VMEM((8, L), jnp.float32),
            pltpu.VMEM((8, L), jnp.int32),
            pltpu.SemaphoreType.DMA,
            pltpu.SemaphoreType.DMA,
            pltpu.SemaphoreType.DMA,
        ],
        compiler_params=pltpu.CompilerParams(needs_layout_passes=False),
    )


def _merge_body(vals_ref, idx_ref, ent_ref, out_ref, row_buf, sem):
    vals = vals_ref[...]
    idx = idx_ref[...]
    m = jnp.max(vals)
    big = jnp.int32(jnp.iinfo(jnp.int32).max)
    best = jnp.min(jnp.where(vals >= m, idx, big))
    base8 = pl.multiple_of((best // 8) * 8, 8)
    copy = pltpu.make_async_copy(ent_ref.at[pl.ds(base8, 8)], row_buf, sem)
    copy.start()
    copy.wait()
    rows = row_buf[...]
    rsel = jax.lax.broadcasted_iota(jnp.int32, (8, D), 0) == (best % 8)
    out_ref[...] = jnp.sum(jnp.where(rsel, rows, 0.0), axis=0)


_merge = pl.pallas_call(
    _merge_body,
    out_shape=jax.ShapeDtypeStruct((D,), jnp.float32),
    in_specs=[
        pl.BlockSpec(memory_space=pltpu.VMEM),
        pl.BlockSpec(memory_space=pltpu.VMEM),
        pl.BlockSpec(memory_space=pltpu.MemorySpace.HBM),
    ],
    scratch_shapes=[
        pltpu.VMEM((8, D), jnp.float32),
        pltpu.SemaphoreType.DMA,
    ],
)


def kernel(head_entity, question_embedding, entity_embeddings):
    vals, idx = _build_scan()(head_entity, question_embedding,
                              entity_embeddings)
    return _merge(vals, idx, entity_embeddings)
